# Initial kernel scaffold; baseline (speedup 1.0000x reference)
import jax
import jax.numpy as jnp
from jax.experimental import pallas as pl


def kernel(x_n, edge_index_n_n, edge_attr_n_n, W_e, b_e, W_n, b_n):
    raise NotImplementedError("write your pallas kernel here")


# speedup vs baseline: 4.6713x; 4.6713x over previous
import functools

import jax
import jax.numpy as jnp
from jax import lax
from jax.experimental import pallas as pl
from jax.experimental.pallas import tpu as pltpu
from jax.experimental.pallas import tpu_sc as plsc

_NC = 2
_NS = 16
_NW = _NC * _NS
_L = 16
_C = 1000


def _proj_tc(x, w_src, w_dst):
    n, df = x.shape
    de = w_src.shape[1]
    bn = 2000

    def body(x_ref, ws_ref, wd_ref, ps_ref, pd_ref):
        xb = x_ref[...]
        ps_ref[...] = jnp.dot(xb, ws_ref[...], preferred_element_type=jnp.float32)
        pd_ref[...] = jnp.dot(xb, wd_ref[...], preferred_element_type=jnp.float32)

    return pl.pallas_call(
        body,
        grid=(n // bn,),
        in_specs=[
            pl.BlockSpec((bn, df), lambda i: (i, 0)),
            pl.BlockSpec((df, de), lambda i: (0, 0)),
            pl.BlockSpec((df, de), lambda i: (0, 0)),
        ],
        out_specs=[
            pl.BlockSpec((bn, de), lambda i: (i, 0)),
            pl.BlockSpec((bn, de), lambda i: (i, 0)),
        ],
        out_shape=[
            jax.ShapeDtypeStruct((n, de), jnp.float32),
            jax.ShapeDtypeStruct((n, de), jnp.float32),
        ],
    )(x, w_src, w_dst)


def _edge_bias_tc(edge_attr, w_attr, b_e):
    e, de = edge_attr.shape
    be = 16000

    def body(ea_ref, w_ref, b_ref, out_ref):
        out_ref[...] = (
            jnp.dot(ea_ref[...], w_ref[...], preferred_element_type=jnp.float32)
            + b_ref[...]
        )

    return pl.pallas_call(
        body,
        grid=(e // be,),
        in_specs=[
            pl.BlockSpec((be, de), lambda i: (i, 0)),
            pl.BlockSpec((de, de), lambda i: (0, 0)),
            pl.BlockSpec((1, de), lambda i: (0, 0)),
        ],
        out_specs=pl.BlockSpec((be, de), lambda i: (i, 0)),
        out_shape=jax.ShapeDtypeStruct((e, de), jnp.float32),
    )(edge_attr, w_attr, b_e)


def _node_tc(x, agg0, agg1, w_n1, w_n2, b_n):
    n, df = x.shape
    de = agg0.shape[1]
    bn = 2000

    def body(x_ref, a0_ref, a1_ref, w1_ref, w2_ref, b_ref, out_ref):
        agg = a0_ref[...] + a1_ref[...]
        acc = jnp.dot(x_ref[...], w1_ref[...], preferred_element_type=jnp.float32)
        acc = acc + jnp.dot(agg, w2_ref[...], preferred_element_type=jnp.float32)
        out_ref[...] = jnp.maximum(acc + b_ref[...], 0.0)

    return pl.pallas_call(
        body,
        grid=(n // bn,),
        in_specs=[
            pl.BlockSpec((bn, df), lambda i: (i, 0)),
            pl.BlockSpec((bn, de), lambda i: (i, 0)),
            pl.BlockSpec((bn, de), lambda i: (i, 0)),
            pl.BlockSpec((df, df), lambda i: (0, 0)),
            pl.BlockSpec((de, df), lambda i: (0, 0)),
            pl.BlockSpec((1, df), lambda i: (0, 0)),
        ],
        out_specs=pl.BlockSpec((bn, df), lambda i: (i, 0)),
        out_shape=jax.ShapeDtypeStruct((n, df), jnp.float32),
    )(x, agg0, agg1, w_n1, w_n2, b_n)


def _sc_edge_agg(p_src, p_dst, a_e, row, col):
    n = p_src.shape[0]
    e = row.shape[0]
    epw = e // _NW
    nch = epw // _C
    npad = ((n + 8 * _NS - 1) // (8 * _NS)) * (8 * _NS)
    nps = npad // _NS

    mesh = plsc.VectorSubcoreMesh(core_axis_name="c", subcore_axis_name="s")

    @functools.partial(
        pl.kernel,
        out_type=[
            jax.ShapeDtypeStruct((e, _L), jnp.float32),
            jax.ShapeDtypeStruct((_NC, npad, _L), jnp.float32),
        ],
        mesh=mesh,
        compiler_params=pltpu.CompilerParams(use_tc_tiling_on_sc=False),
        scratch_types=[
            pltpu.VMEM((_C,), jnp.int32),
            pltpu.VMEM((_C,), jnp.int32),
            pltpu.VMEM((_C, _L), jnp.float32),
            pltpu.VMEM((_C, _L), jnp.float32),
            pltpu.VMEM((_C, _L), jnp.float32),
            pltpu.VMEM((_C, _L), jnp.float32),
            pltpu.VMEM((nps, _L), jnp.float32),
            pltpu.VMEM_SHARED((npad, _L), jnp.float32),
            pltpu.SemaphoreType.DMA,
            pltpu.SemaphoreType.DMA,
        ],
    )
    def k(ps_h, pd_h, a_h, row_h, col_h, eo_h, agg_h,
          row_v, col_v, src_v, dst_v, a_v, res_v, zb_v, acc_s, sem0, sem1):
        cid = lax.axis_index("c")
        sid = lax.axis_index("s")
        wid = cid * _NS + sid
        ebase = wid * epw

        def _z(i, carry):
            zb_v[i] = jnp.zeros((_L,), jnp.float32)
            return carry

        lax.fori_loop(0, nps, _z, 0)
        pltpu.sync_copy(zb_v, acc_s.at[pl.ds(sid * nps, nps)])
        plsc.subcore_barrier()

        def _chunk(ci, carry):
            base = pl.multiple_of(ebase + ci * _C, 8)
            pltpu.sync_copy(row_h.at[pl.ds(base, _C)], row_v)
            pltpu.sync_copy(col_h.at[pl.ds(base, _C)], col_v)
            pltpu.sync_copy(a_h.at[pl.ds(base, _C)], a_v)
            pltpu.async_copy(ps_h.at[row_v], src_v, sem0).wait()
            pltpu.async_copy(pd_h.at[col_v], dst_v, sem1).wait()

            def _edge(i, c2):
                res_v[i] = jnp.maximum(src_v[i] + dst_v[i] + a_v[i], 0.0)
                return c2

            lax.fori_loop(0, _C, _edge, 0)
            pltpu.sync_copy(res_v, eo_h.at[pl.ds(base, _C)])
            pltpu.sync_copy(res_v, acc_s.at[col_v], add=True)
            return carry

        lax.fori_loop(0, nch, _chunk, 0)
        plsc.subcore_barrier()
        pltpu.sync_copy(
            acc_s.at[pl.ds(sid * nps, nps)],
            agg_h.at[cid, pl.ds(sid * nps, nps)],
        )

    return k(p_src, p_dst, a_e, row, col)


def kernel(x_n, edge_index_n_n, edge_attr_n_n, W_e, b_e, W_n, b_n):
    x = x_n.astype(jnp.float32)
    df = x.shape[1]
    de = edge_attr_n_n.shape[1]
    row = edge_index_n_n[0].astype(jnp.int32)
    col = edge_index_n_n[1].astype(jnp.int32)

    w_src = W_e[:df]
    w_dst = W_e[df:2 * df]
    w_attr = W_e[2 * df:]

    p_src, p_dst = _proj_tc(x, w_src, w_dst)
    a_e = _edge_bias_tc(edge_attr_n_n.astype(jnp.float32), w_attr,
                        b_e.reshape(1, de))
    n = x.shape[0]
    new_edge_attr, agg_parts = _sc_edge_agg(p_src, p_dst, a_e, row, col)
    new_x = _node_tc(x, agg_parts[0, :n], agg_parts[1, :n],
                     W_n[:df], W_n[df:], b_n.reshape(1, df))
    return (new_x, new_edge_attr)
